# baseline (speedup 1.0000x reference)
import functools

import jax
import jax.numpy as jnp
from jax import lax
from jax.experimental import pallas as pl
from jax.experimental.pallas import tpu as pltpu
from jax.experimental.pallas import tpu_sc as plsc

VOCAB = 100000
BATCH = 32
Q_LEN = 8
ROWS = BATCH * Q_LEN
BB = 2


def _fill_body(ls_ref, ids_ref, out_ref):
    i = pl.program_id(0)
    alpha = jax.nn.sigmoid(ls_ref[pl.ds(i * BB, BB), :])
    base = (1.0 - alpha) * jnp.float32(1.0 / VOCAB)
    log_base = jnp.maximum(jnp.log(base), jnp.float32(-1e6))
    log_peak = jnp.maximum(jnp.log(base + alpha), jnp.float32(-1e6))
    col = jax.lax.broadcasted_iota(jnp.int32, (BB, Q_LEN, VOCAB), 2)
    mask = col == ids_ref[pl.ds(i * BB, BB), :][..., None]
    out_ref[...] = jnp.where(mask, log_peak[..., None], log_base[..., None])


def _sc_probe(out_ref, probe_v, sem):
    wid = lax.axis_index("s") * 2 + lax.axis_index("c")

    @pl.when(wid == 0)
    def _():
        pltpu.sync_copy(out_ref.at[0, 0, pl.ds(0, 16)], probe_v)
        pltpu.sync_copy(probe_v, out_ref.at[0, 0, pl.ds(0, 16)])


@jax.jit
def kernel(log_snr, input_ids):
    filled = pl.pallas_call(
        _fill_body,
        grid=(BATCH // BB,),
        in_specs=[
            pl.BlockSpec((BATCH, Q_LEN), lambda i: (0, 0)),
            pl.BlockSpec((BATCH, Q_LEN), lambda i: (0, 0)),
        ],
        out_specs=pl.BlockSpec((BB, Q_LEN, VOCAB), lambda i: (i, 0, 0)),
        out_shape=jax.ShapeDtypeStruct((BATCH, Q_LEN, VOCAB), jnp.float32),
    )(log_snr, input_ids.astype(jnp.int32))
    out_ref = jax.new_ref(filled)
    mesh = plsc.VectorSubcoreMesh(core_axis_name="c", subcore_axis_name="s")
    probe = functools.partial(
        pl.kernel,
        mesh=mesh,
        scratch_types=[
            pltpu.VMEM((16,), jnp.float32),
            pltpu.SemaphoreType.DMA,
        ],
    )(_sc_probe)
    probe(out_ref)
    return jax.freeze(out_ref)
